# baseline (speedup 1.0000x reference)
import functools

import jax
import jax.numpy as jnp
from jax import lax
from jax.experimental import pallas as pl
from jax.experimental.pallas import tpu as pltpu
from jax.experimental.pallas import tpu_sc as plsc

_N = 50000
_M = 50000
_D = 64
_E = 800000
_B = 4096

_NC = 2
_NS = 16
_H = _D // _NC
_NP = 50048
_SUB = 128
_CPS = 2
_CHUNK = _SUB * _CPS
_NCHUNK = -(-_E // (_NS * _CHUNK))
_EPT = _NCHUNK * _CHUNK
_EPAD = _NS * _EPT
_ERPT = _EPT // _SUB
_RPT = _NP // _NS
_ZR = 184


def _body(u0, i0, vals, gxu, gxi, sxu, sxi, uid2, iid2,
          uo, io, u1, u2, u3, i1, i2, i3,
          acc, zb, fi, gi, si, vb, gb, sem_l, sem_g, sem_s):
  c = lax.axis_index("c")
  s = lax.axis_index("s")

  @plsc.parallel_loop(0, _ZR, 1, unroll=4)
  def _(r):
    zb[r, pl.ds(0, 16)] = jnp.zeros((16,), jnp.float32)
    zb[r, pl.ds(16, 16)] = jnp.zeros((16,), jnp.float32)

  def propagate(src_tab, gx, sx, out_tab):
    base = s * _RPT
    zd = [
        pltpu.async_copy(zb, acc.at[pl.ds(base + t * _ZR, _ZR)], sem_l)
        for t in range(_RPT // _ZR)
    ]
    for d in zd:
      d.wait()
    plsc.subcore_barrier()

    g_base = c * _EPAD + s * _EPT
    s_base = s * _ERPT
    v_base = s * _EPT

    def idx_start(k):
      p = lax.rem(k, 2)
      s3 = lax.rem(k, 3)
      pltpu.async_copy(
          gx.at[pl.ds(g_base + k * _CHUNK, _CHUNK)], gi.at[p], sem_l)
      pltpu.async_copy(
          sx.at[pl.ds(s_base + k * _CPS, _CPS)], si.at[s3], sem_l)
      pltpu.async_copy(
          vals.at[pl.ds(v_base + k * _CHUNK, _CHUNK)], vb.at[p], sem_l)

    def idx_wait(k):
      p = lax.rem(k, 2)
      s3 = lax.rem(k, 3)
      pltpu.make_async_copy(
          gx.at[pl.ds(g_base + k * _CHUNK, _CHUNK)], gi.at[p], sem_l).wait()
      pltpu.make_async_copy(
          sx.at[pl.ds(s_base + k * _CPS, _CPS)], si.at[s3], sem_l).wait()
      pltpu.make_async_copy(
          vals.at[pl.ds(v_base + k * _CHUNK, _CHUNK)], vb.at[p], sem_l).wait()

    def gather_start(k):
      p = lax.rem(k, 2)
      for q in range(_CPS):
        pltpu.async_copy(
            src_tab.at[gi.at[p, pl.ds(q * _SUB, _SUB)]],
            gb.at[p, pl.ds(q * _SUB, _SUB)], sem_g)

    def gather_wait(k):
      p = lax.rem(k, 2)
      for q in range(_CPS):
        pltpu.make_async_copy(
            src_tab.at[gi.at[p, pl.ds(q * _SUB, _SUB)]],
            gb.at[p, pl.ds(q * _SUB, _SUB)], sem_g).wait()

    def scatter_start(k):
      p = lax.rem(k, 2)
      s3 = lax.rem(k, 3)
      for q in range(_CPS):
        pltpu.async_copy(
            gb.at[p, pl.ds(q * _SUB, _SUB)],
            acc.at[si.at[s3, q]], sem_s, add=True)

    def scatter_wait(k):
      p = lax.rem(k, 2)
      s3 = lax.rem(k, 3)
      for q in range(_CPS):
        pltpu.make_async_copy(
            gb.at[p, pl.ds(q * _SUB, _SUB)],
            acc.at[si.at[s3, q]], sem_s).wait()

    idx_start(jnp.int32(0))
    idx_wait(jnp.int32(0))
    gather_start(jnp.int32(0))
    idx_start(jnp.int32(1))

    def chunk_body(k, carry):
      p = lax.rem(k, 2)

      @pl.when(k + 1 < _NCHUNK)
      def _():
        idx_wait(k + 1)

      @pl.when(k >= 1)
      def _():
        scatter_wait(k - 1)

      @pl.when(k + 1 < _NCHUNK)
      def _():
        gather_start(k + 1)

      gather_wait(k)

      @plsc.parallel_loop(0, _CHUNK, 16, unroll=4)
      def _(e0):
        vv = vb[p, pl.ds(e0, 16)]
        for l in range(16):
          v = vv[l]
          gb[p, e0 + l, pl.ds(0, 16)] = gb[p, e0 + l, pl.ds(0, 16)] * v
          gb[p, e0 + l, pl.ds(16, 16)] = gb[p, e0 + l, pl.ds(16, 16)] * v

      scatter_start(k)

      @pl.when(k + 2 < _NCHUNK)
      def _():
        idx_start(k + 2)

      return carry

    lax.fori_loop(0, _NCHUNK, chunk_body, 0)
    scatter_wait(jnp.int32(_NCHUNK - 1))
    plsc.subcore_barrier()
    dd = [
        pltpu.async_copy(acc.at[pl.ds(base + t * _ZR, _ZR)],
                         out_tab.at[pl.ds(c * _NP + base + t * _ZR, _ZR)],
                         sem_l)
        for t in range(_RPT // _ZR)
    ]
    for d in dd:
      d.wait()
    plsc.subcore_barrier()

  propagate(i0, gxu, sxu, u1)
  propagate(u0, gxi, sxi, i1)
  propagate(i1, gxu, sxu, u2)
  propagate(u1, gxi, sxi, i2)
  propagate(i2, gxu, sxu, u3)
  propagate(u2, gxi, sxi, i3)

  def final_side(t0, t1, t2, t3, id2, out):
    pltpu.sync_copy(id2.at[pl.ds(c * _B + s * 256, 256)], fi)
    for r in range(2):
      gd = [
          pltpu.async_copy(tab.at[fi.at[pl.ds(r * _SUB, _SUB)]],
                           gb.at[0, pl.ds(t * _SUB, _SUB)]
                           if t < _CPS else
                           gb.at[1, pl.ds((t - _CPS) * _SUB, _SUB)], sem_g)
          for t, tab in enumerate((t0, t1, t2, t3))
      ]
      for d in gd:
        d.wait()

      @plsc.parallel_loop(0, _SUB, 1, unroll=4)
      def _(e):
        for h in range(2):
          sl = pl.ds(h * 16, 16)
          zb[e, sl] = (gb[0, e, sl] + gb[0, _SUB + e, sl]
                       + gb[1, e, sl] + gb[1, _SUB + e, sl]) * 0.25

      pltpu.sync_copy(
          zb.at[pl.ds(0, _SUB)],
          out.at[pl.ds(c * _B + s * 256 + r * _SUB, _SUB)])

  final_side(u0, u1, u2, u3, uid2, uo)
  final_side(i0, i1, i2, i3, iid2, io)


_TAB = jax.ShapeDtypeStruct((_NC * _NP, _H), jnp.float32)
_OUT = jax.ShapeDtypeStruct((_NC * _B, _H), jnp.float32)

_sc_call = functools.partial(
    pl.kernel,
    out_type=[_OUT, _OUT, _TAB, _TAB, _TAB, _TAB, _TAB, _TAB],
    mesh=plsc.VectorSubcoreMesh(
        core_axis_name="c", subcore_axis_name="s",
        num_cores=_NC, num_subcores=_NS),
    compiler_params=pltpu.CompilerParams(use_tc_tiling_on_sc=False),
    scratch_types=[
        pltpu.VMEM_SHARED((_NP, _H), jnp.float32),
        pltpu.VMEM((_ZR, _H), jnp.float32),
        pltpu.VMEM((256,), jnp.int32),
        pltpu.VMEM((2, _CHUNK), jnp.int32),
        pltpu.VMEM((3, _CPS, _SUB), jnp.int32),
        pltpu.VMEM((2, _CHUNK), jnp.float32),
        pltpu.VMEM((2, _CHUNK, _H), jnp.float32),
        pltpu.SemaphoreType.DMA,
        pltpu.SemaphoreType.DMA,
        pltpu.SemaphoreType.DMA,
    ],
)(_body)


def kernel(user_embeddings, item_embeddings, A_vals, A_rows, A_cols,
           u_id, i_id):
  rows = A_rows.astype(jnp.int32)
  cols = A_cols.astype(jnp.int32)
  vals = A_vals.astype(jnp.float32)
  pad = _EPAD - _E
  rows_p = jnp.concatenate([rows, jnp.zeros((pad,), jnp.int32)])
  cols_p = jnp.concatenate([cols, jnp.zeros((pad,), jnp.int32)])
  vals_p = jnp.concatenate([vals, jnp.zeros((pad,), jnp.float32)])

  gxu = jnp.concatenate([cols_p, cols_p + _NP])
  gxi = jnp.concatenate([rows_p, rows_p + _NP])
  sxu = rows_p.reshape(-1, _SUB)
  sxi = cols_p.reshape(-1, _SUB)

  uid = u_id.astype(jnp.int32)
  iid = i_id.astype(jnp.int32)
  uid2 = jnp.concatenate([uid, uid + _NP])
  iid2 = jnp.concatenate([iid, iid + _NP])

  u0 = jnp.pad(
      user_embeddings.reshape(_N, _NC, _H).transpose(1, 0, 2),
      ((0, 0), (0, _NP - _N), (0, 0))).reshape(_NC * _NP, _H)
  i0 = jnp.pad(
      item_embeddings.reshape(_M, _NC, _H).transpose(1, 0, 2),
      ((0, 0), (0, _NP - _M), (0, 0))).reshape(_NC * _NP, _H)

  uo, io, *_ = _sc_call(u0, i0, vals_p, gxu, gxi, sxu, sxi, uid2, iid2)

  user_forward = uo.reshape(_NC, _B, _H).transpose(1, 0, 2).reshape(_B, _D)
  item_forward = io.reshape(_NC, _B, _H).transpose(1, 0, 2).reshape(_B, _D)
  return (user_forward, item_forward)


# speedup vs baseline: 1.0305x; 1.0305x over previous
import functools

import jax
import jax.numpy as jnp
from jax import lax
from jax.experimental import pallas as pl
from jax.experimental.pallas import tpu as pltpu
from jax.experimental.pallas import tpu_sc as plsc

_N = 50000
_M = 50000
_D = 64
_E = 800000
_B = 4096

_NC = 2
_NS = 16
_H = _D // _NC
_NP = 50048
_SUB = 128
_CPS = 2
_CHUNK = _SUB * _CPS
_NCHUNK = -(-_E // (_NS * _CHUNK))
_EPT = _NCHUNK * _CHUNK
_EPAD = _NS * _EPT
_ERPT = _EPT // _SUB
_RPT = _NP // _NS
_ZR = 184


def _body(u0, i0, vals, gxu, gxi, sxu, sxi, uid2, iid2,
          uo, io, u1, u2, u3, i1, i2, i3,
          acc, zb, fi, gi, si, vb, gb, sem_l, sem_g, sem_s):
  c = lax.axis_index("c")
  s = lax.axis_index("s")

  @plsc.parallel_loop(0, _ZR, 1, unroll=4)
  def _(r):
    zb[r, pl.ds(0, 16)] = jnp.zeros((16,), jnp.float32)
    zb[r, pl.ds(16, 16)] = jnp.zeros((16,), jnp.float32)

  def propagate(src_tab, gx, sx, out_tab):
    base = s * _RPT
    zd = [
        pltpu.async_copy(zb, acc.at[pl.ds(base + t * _ZR, _ZR)], sem_l)
        for t in range(_RPT // _ZR)
    ]
    for d in zd:
      d.wait()
    plsc.subcore_barrier()

    g_base = c * _EPAD + s * _EPT
    s_base = s * _ERPT
    v_base = s * _EPT

    def idx_start(k):
      p = lax.rem(k, 2)
      s3 = lax.rem(k, 3)
      pltpu.async_copy(
          gx.at[pl.ds(g_base + k * _CHUNK, _CHUNK)], gi.at[p], sem_l)
      pltpu.async_copy(
          sx.at[pl.ds(s_base + k * _CPS, _CPS)], si.at[s3], sem_l)
      pltpu.async_copy(
          vals.at[pl.ds(v_base + k * _CHUNK, _CHUNK)], vb.at[p], sem_l)

    def idx_wait(k):
      p = lax.rem(k, 2)
      s3 = lax.rem(k, 3)
      pltpu.make_async_copy(
          gx.at[pl.ds(g_base + k * _CHUNK, _CHUNK)], gi.at[p], sem_l).wait()
      pltpu.make_async_copy(
          sx.at[pl.ds(s_base + k * _CPS, _CPS)], si.at[s3], sem_l).wait()
      pltpu.make_async_copy(
          vals.at[pl.ds(v_base + k * _CHUNK, _CHUNK)], vb.at[p], sem_l).wait()

    def gather_start(k):
      p = lax.rem(k, 2)
      for q in range(_CPS):
        pltpu.async_copy(
            src_tab.at[gi.at[p, pl.ds(q * _SUB, _SUB)]],
            gb.at[p, pl.ds(q * _SUB, _SUB)], sem_g)

    def gather_wait(k):
      p = lax.rem(k, 2)
      for q in range(_CPS):
        pltpu.make_async_copy(
            src_tab.at[gi.at[p, pl.ds(q * _SUB, _SUB)]],
            gb.at[p, pl.ds(q * _SUB, _SUB)], sem_g).wait()

    def scatter_start(k):
      p = lax.rem(k, 2)
      s3 = lax.rem(k, 3)
      for q in range(_CPS):
        pltpu.async_copy(
            gb.at[p, pl.ds(q * _SUB, _SUB)],
            acc.at[si.at[s3, q]], sem_s, add=True)

    def scatter_wait(k):
      p = lax.rem(k, 2)
      s3 = lax.rem(k, 3)
      for q in range(_CPS):
        pltpu.make_async_copy(
            gb.at[p, pl.ds(q * _SUB, _SUB)],
            acc.at[si.at[s3, q]], sem_s).wait()

    idx_start(jnp.int32(0))
    idx_wait(jnp.int32(0))
    gather_start(jnp.int32(0))
    idx_start(jnp.int32(1))

    def chunk_body(k, carry):
      p = lax.rem(k, 2)

      @pl.when(k + 1 < _NCHUNK)
      def _():
        idx_wait(k + 1)

      @pl.when(k >= 1)
      def _():
        scatter_wait(k - 1)

      @pl.when(k + 1 < _NCHUNK)
      def _():
        gather_start(k + 1)

      gather_wait(k)

      @plsc.parallel_loop(0, _CHUNK, 16, unroll=2)
      def _(e0):
        vv = vb[p, pl.ds(e0, 16)]
        for l in range(16):
          v = vv[l]
          gb[p, e0 + l, pl.ds(0, 16)] = gb[p, e0 + l, pl.ds(0, 16)] * v
          gb[p, e0 + l, pl.ds(16, 16)] = gb[p, e0 + l, pl.ds(16, 16)] * v

      scatter_start(k)

      @pl.when(k + 2 < _NCHUNK)
      def _():
        idx_start(k + 2)

      return carry

    lax.fori_loop(0, _NCHUNK, chunk_body, 0)
    scatter_wait(jnp.int32(_NCHUNK - 1))
    plsc.subcore_barrier()
    dd = [
        pltpu.async_copy(acc.at[pl.ds(base + t * _ZR, _ZR)],
                         out_tab.at[pl.ds(c * _NP + base + t * _ZR, _ZR)],
                         sem_l)
        for t in range(_RPT // _ZR)
    ]
    for d in dd:
      d.wait()
    plsc.subcore_barrier()

  propagate(i0, gxu, sxu, u1)
  propagate(u0, gxi, sxi, i1)
  propagate(i1, gxu, sxu, u2)
  propagate(u1, gxi, sxi, i2)
  propagate(i2, gxu, sxu, u3)
  propagate(u2, gxi, sxi, i3)

  def final_side(t0, t1, t2, t3, id2, out):
    pltpu.sync_copy(id2.at[pl.ds(c * _B + s * 256, 256)], fi)
    for r in range(2):
      gd = [
          pltpu.async_copy(tab.at[fi.at[pl.ds(r * _SUB, _SUB)]],
                           gb.at[0, pl.ds(t * _SUB, _SUB)]
                           if t < _CPS else
                           gb.at[1, pl.ds((t - _CPS) * _SUB, _SUB)], sem_g)
          for t, tab in enumerate((t0, t1, t2, t3))
      ]
      for d in gd:
        d.wait()

      @plsc.parallel_loop(0, _SUB, 1, unroll=4)
      def _(e):
        for h in range(2):
          sl = pl.ds(h * 16, 16)
          zb[e, sl] = (gb[0, e, sl] + gb[0, _SUB + e, sl]
                       + gb[1, e, sl] + gb[1, _SUB + e, sl]) * 0.25

      pltpu.sync_copy(
          zb.at[pl.ds(0, _SUB)],
          out.at[pl.ds(c * _B + s * 256 + r * _SUB, _SUB)])

  final_side(u0, u1, u2, u3, uid2, uo)
  final_side(i0, i1, i2, i3, iid2, io)


_TAB = jax.ShapeDtypeStruct((_NC * _NP, _H), jnp.float32)
_OUT = jax.ShapeDtypeStruct((_NC * _B, _H), jnp.float32)

_sc_call = functools.partial(
    pl.kernel,
    out_type=[_OUT, _OUT, _TAB, _TAB, _TAB, _TAB, _TAB, _TAB],
    mesh=plsc.VectorSubcoreMesh(
        core_axis_name="c", subcore_axis_name="s",
        num_cores=_NC, num_subcores=_NS),
    compiler_params=pltpu.CompilerParams(use_tc_tiling_on_sc=False),
    scratch_types=[
        pltpu.VMEM_SHARED((_NP, _H), jnp.float32),
        pltpu.VMEM((_ZR, _H), jnp.float32),
        pltpu.VMEM((256,), jnp.int32),
        pltpu.VMEM((2, _CHUNK), jnp.int32),
        pltpu.VMEM((3, _CPS, _SUB), jnp.int32),
        pltpu.VMEM((2, _CHUNK), jnp.float32),
        pltpu.VMEM((2, _CHUNK, _H), jnp.float32),
        pltpu.SemaphoreType.DMA,
        pltpu.SemaphoreType.DMA,
        pltpu.SemaphoreType.DMA,
    ],
)(_body)


def kernel(user_embeddings, item_embeddings, A_vals, A_rows, A_cols,
           u_id, i_id):
  rows = A_rows.astype(jnp.int32)
  cols = A_cols.astype(jnp.int32)
  vals = A_vals.astype(jnp.float32)
  pad = _EPAD - _E
  rows_p = jnp.concatenate([rows, jnp.zeros((pad,), jnp.int32)])
  cols_p = jnp.concatenate([cols, jnp.zeros((pad,), jnp.int32)])
  vals_p = jnp.concatenate([vals, jnp.zeros((pad,), jnp.float32)])

  gxu = jnp.concatenate([cols_p, cols_p + _NP])
  gxi = jnp.concatenate([rows_p, rows_p + _NP])
  sxu = rows_p.reshape(-1, _SUB)
  sxi = cols_p.reshape(-1, _SUB)

  uid = u_id.astype(jnp.int32)
  iid = i_id.astype(jnp.int32)
  uid2 = jnp.concatenate([uid, uid + _NP])
  iid2 = jnp.concatenate([iid, iid + _NP])

  u0 = jnp.pad(
      user_embeddings.reshape(_N, _NC, _H).transpose(1, 0, 2),
      ((0, 0), (0, _NP - _N), (0, 0))).reshape(_NC * _NP, _H)
  i0 = jnp.pad(
      item_embeddings.reshape(_M, _NC, _H).transpose(1, 0, 2),
      ((0, 0), (0, _NP - _M), (0, 0))).reshape(_NC * _NP, _H)

  uo, io, *_ = _sc_call(u0, i0, vals_p, gxu, gxi, sxu, sxi, uid2, iid2)

  user_forward = uo.reshape(_NC, _B, _H).transpose(1, 0, 2).reshape(_B, _D)
  item_forward = io.reshape(_NC, _B, _H).transpose(1, 0, 2).reshape(_B, _D)
  return (user_forward, item_forward)
